# Initial kernel scaffold; baseline (speedup 1.0000x reference)
import jax
import jax.numpy as jnp
from jax.experimental import pallas as pl


def kernel(position_ids, embeddings_table):
    raise NotImplementedError("write your pallas kernel here")


# speedup vs baseline: 1.9961x; 1.9961x over previous
import functools

import jax
import jax.numpy as jnp
from jax import lax
from jax.experimental import pallas as pl
from jax.experimental.pallas import tpu as pltpu
from jax.experimental.pallas import tpu_sc as plsc

EMB = 1024
NC = 2
NS = 16
NW = NC * NS

B_TOTAL = 4 * 8192
B_PER_W = B_TOTAL // NW
CHUNK = 32
N_CHUNKS = B_PER_W // CHUNK


def _gather_body(idx_hbm, table_hbm, out_hbm, idx_v, buf_v, insem):
    wid = lax.axis_index("s") * NC + lax.axis_index("c")
    base = wid * B_PER_W
    pltpu.sync_copy(idx_hbm.at[pl.ds(wid * N_CHUNKS, N_CHUNKS)], idx_v)

    def chunk_step(c, carry):
        pltpu.async_copy(table_hbm.at[idx_v.at[c]], buf_v, insem).wait()
        pltpu.sync_copy(buf_v, out_hbm.at[pl.ds(base + c * CHUNK, CHUNK)])
        return carry

    lax.fori_loop(0, N_CHUNKS, chunk_step, 0)


@functools.partial(jax.jit, static_argnums=())
def _gather_call(idx2d, table):
    mesh = plsc.VectorSubcoreMesh(
        core_axis_name="c", subcore_axis_name="s",
        num_cores=NC, num_subcores=NS)
    return pl.kernel(
        _gather_body,
        out_type=jax.ShapeDtypeStruct((B_TOTAL, EMB), jnp.float32),
        mesh=mesh,
        scratch_types=[
            pltpu.VMEM((N_CHUNKS, CHUNK), jnp.int32),
            pltpu.VMEM((CHUNK, EMB), jnp.float32),
            pltpu.SemaphoreType.DMA,
        ],
    )(idx2d, table)


def kernel(position_ids, embeddings_table):
    batch, seq = position_ids.shape
    idx2d = position_ids.reshape(B_TOTAL // CHUNK, CHUNK)
    out = _gather_call(idx2d, embeddings_table)
    return out.reshape(batch, seq, EMB)


# speedup vs baseline: 2.2942x; 1.1494x over previous
import functools

import jax
import jax.numpy as jnp
from jax import lax
from jax.experimental import pallas as pl
from jax.experimental.pallas import tpu as pltpu
from jax.experimental.pallas import tpu_sc as plsc

EMB = 1024
NC = 2
NS = 16
NW = NC * NS

B_TOTAL = 4 * 8192
B_PER_W = B_TOTAL // NW
CHUNK = 32
N_CHUNKS = B_PER_W // CHUNK


def _gather_body(idx_hbm, table_hbm, out_hbm,
                 idx_v, buf0, buf1, in0, in1, out0, out1):
    wid = lax.axis_index("s") * NC + lax.axis_index("c")
    base = wid * B_PER_W
    bufs = (buf0, buf1)
    insems = (in0, in1)
    outsems = (out0, out1)

    pltpu.sync_copy(idx_hbm.at[pl.ds(wid * N_CHUNKS, N_CHUNKS)], idx_v)

    def start_in(b, g):
        pltpu.async_copy(table_hbm.at[idx_v.at[g]], bufs[b], insems[b])

    def wait_in(b):
        pltpu.make_async_copy(table_hbm.at[idx_v.at[0]], bufs[b],
                              insems[b]).wait()

    def start_out(b, g):
        pltpu.async_copy(bufs[b], out_hbm.at[pl.ds(base + g * CHUNK, CHUNK)],
                         outsems[b])

    def wait_out(b):
        pltpu.make_async_copy(out_hbm.at[pl.ds(base, CHUNK)], bufs[b],
                              outsems[b]).wait()

    start_in(0, 0)
    wait_in(0)
    start_in(1, 1)
    start_out(0, 0)
    wait_in(1)
    wait_out(0)
    start_in(0, 2)
    start_out(1, 1)

    def pair_step(p, carry):
        g = 2 * p
        wait_in(0)
        wait_out(1)
        start_in(1, g + 1)
        start_out(0, g)
        wait_in(1)
        wait_out(0)
        start_in(0, g + 2)
        start_out(1, g + 1)
        return carry

    lax.fori_loop(1, N_CHUNKS // 2 - 1, pair_step, 0)

    g = N_CHUNKS - 2
    wait_in(0)
    wait_out(1)
    start_in(1, g + 1)
    start_out(0, g)
    wait_in(1)
    wait_out(0)
    start_out(1, g + 1)
    wait_out(1)


@jax.jit
def _gather_call(idx2d, table):
    mesh = plsc.VectorSubcoreMesh(
        core_axis_name="c", subcore_axis_name="s",
        num_cores=NC, num_subcores=NS)
    return pl.kernel(
        _gather_body,
        out_type=jax.ShapeDtypeStruct((B_TOTAL, EMB), jnp.float32),
        mesh=mesh,
        scratch_types=[
            pltpu.VMEM((N_CHUNKS, CHUNK), jnp.int32),
            pltpu.VMEM((CHUNK, EMB), jnp.float32),
            pltpu.VMEM((CHUNK, EMB), jnp.float32),
            pltpu.SemaphoreType.DMA,
            pltpu.SemaphoreType.DMA,
            pltpu.SemaphoreType.DMA,
            pltpu.SemaphoreType.DMA,
        ],
    )(idx2d, table)


def kernel(position_ids, embeddings_table):
    batch, seq = position_ids.shape
    idx2d = position_ids.reshape(B_TOTAL // CHUNK, CHUNK)
    out = _gather_call(idx2d, embeddings_table)
    return out.reshape(batch, seq, EMB)


# speedup vs baseline: 2.3908x; 1.0421x over previous
import jax
import jax.numpy as jnp
from jax import lax
from jax.experimental import pallas as pl
from jax.experimental.pallas import tpu as pltpu
from jax.experimental.pallas import tpu_sc as plsc

EMB = 1024
NC = 2
NS = 16
NW = NC * NS

B_TOTAL = 4 * 8192
B_PER_W = B_TOTAL // NW
CHUNK = 32
N_CHUNKS = B_PER_W // CHUNK
NBUF = 3


def _gather_body(idx_hbm, table_hbm, out_hbm,
                 idx_v, buf0, buf1, buf2, in0, in1, in2, out0, out1, out2):
    wid = lax.axis_index("s") * NC + lax.axis_index("c")
    base = wid * B_PER_W
    bufs = (buf0, buf1, buf2)
    insems = (in0, in1, in2)
    outsems = (out0, out1, out2)

    pltpu.sync_copy(idx_hbm.at[pl.ds(wid * N_CHUNKS, N_CHUNKS)], idx_v)

    def start_in(b, g):
        pltpu.async_copy(table_hbm.at[idx_v.at[g]], bufs[b], insems[b])

    def wait_in(b):
        pltpu.make_async_copy(table_hbm.at[idx_v.at[0]], bufs[b],
                              insems[b]).wait()

    def start_out(b, g):
        pltpu.async_copy(bufs[b], out_hbm.at[pl.ds(base + g * CHUNK, CHUNK)],
                         outsems[b])

    def wait_out(b):
        pltpu.make_async_copy(out_hbm.at[pl.ds(base, CHUNK)], bufs[b],
                              outsems[b]).wait()

    def emit(g, b, first=False, startin=True):
        wait_in(b)
        if not first:
            wait_out((b + NBUF - 1) % NBUF)
        start_out(b, g)
        if startin:
            start_in((b + NBUF - 1) % NBUF, g + NBUF - 1)

    start_in(0, 0)
    start_in(1, 1)
    emit(0, 0, first=True)

    def triple_step(p, carry):
        g0 = 1 + 3 * p
        emit(g0, 1)
        emit(g0 + 1, 2)
        emit(g0 + 2, 0)
        return carry

    lax.fori_loop(0, 9, triple_step, 0)
    emit(28, 1)
    emit(29, 2)
    emit(30, 0, startin=False)
    emit(31, 1, startin=False)
    wait_out(1)


@jax.jit
def _gather_call(idx2d, table):
    mesh = plsc.VectorSubcoreMesh(
        core_axis_name="c", subcore_axis_name="s",
        num_cores=NC, num_subcores=NS)
    return pl.kernel(
        _gather_body,
        out_type=jax.ShapeDtypeStruct((B_TOTAL, EMB), jnp.float32),
        mesh=mesh,
        scratch_types=[
            pltpu.VMEM((N_CHUNKS, CHUNK), jnp.int32),
            pltpu.VMEM((CHUNK, EMB), jnp.float32),
            pltpu.VMEM((CHUNK, EMB), jnp.float32),
            pltpu.VMEM((CHUNK, EMB), jnp.float32),
            pltpu.SemaphoreType.DMA,
            pltpu.SemaphoreType.DMA,
            pltpu.SemaphoreType.DMA,
            pltpu.SemaphoreType.DMA,
            pltpu.SemaphoreType.DMA,
            pltpu.SemaphoreType.DMA,
        ],
    )(idx2d, table)


def kernel(position_ids, embeddings_table):
    batch, seq = position_ids.shape
    idx2d = position_ids.reshape(B_TOTAL // CHUNK, CHUNK)
    out = _gather_call(idx2d, embeddings_table)
    return out.reshape(batch, seq, EMB)
